# Initial kernel scaffold; baseline (speedup 1.0000x reference)
import jax
import jax.numpy as jnp
from jax.experimental import pallas as pl


def kernel(latent, action, W_mask, b_mask, W1, b1, W2, b2):
    raise NotImplementedError("write your pallas kernel here")


# speedup vs baseline: 3.8395x; 3.8395x over previous
import math

import jax
import jax.numpy as jnp
import numpy as np
from jax.experimental import pallas as pl
from jax.experimental.pallas import tpu as pltpu


def _pos_encoding_np(N, D):
    position = np.arange(N)[:, None].astype(np.float64)
    div_term = np.exp(np.arange(0, D, 2).astype(np.float64) * (-math.log(10000.0) / D))
    pe = np.zeros((N, D), dtype=np.float32)
    pe[:, 0::2] = np.sin(position * div_term)
    pe[:, 1::2] = np.cos(position * div_term)
    return pe


def _adj_body(ids_ref, latent_ref, action_ref, wmask_ref, bmask_ref, pos_ref,
              g_ref, w10_ref, w1e_ref, b10_ref, b1e_ref, w20_ref, w2e_ref,
              b20_ref, b2e_ref, out_ref):
    N, D = latent_ref.shape[-2], latent_ref.shape[-1]
    A = action_ref.shape[-1]
    lat = latent_ref[0]
    act = action_ref[0]

    act_rep = jnp.broadcast_to(act, (N, A))
    ap = jnp.concatenate([act_rep, pos_ref[...]], axis=-1)
    inter_mask = jax.nn.sigmoid(
        jnp.dot(ap, wmask_ref[...], preferred_element_type=jnp.float32)
        + bmask_ref[...])
    s = jnp.sum(lat * inter_mask, axis=-1, keepdims=True)
    l0 = jnp.log(jnp.maximum(1.0 - s, 0.0001))
    l1 = jnp.log(jnp.maximum(s, 0.0001))
    g = g_ref[0]
    z0 = l0 + g[:, 0:1]
    z1 = l1 + g[:, 1:2]
    zm = jnp.maximum(z0, z1)
    e0 = jnp.exp(z0 - zm)
    e1 = jnp.exp(z1 - zm)
    esum = e0 + e1
    y0 = e0 / esum
    y1 = e1 / esum
    hard = (y1 > y0).astype(jnp.float32)
    mask = hard + y1 - y1

    def expert(w1_ref, b1_ref, w2_ref, b2_ref):
        w1 = w1_ref[0]
        H = w1.shape[-1]
        u = jnp.dot(lat, w1[:D], preferred_element_type=jnp.float32)
        v = jnp.dot(lat, w1[D:], preferred_element_type=jnp.float32)
        big = u[:, None, :] + v[None, :, :] + b1_ref[0][None, :, :]
        h = jnp.maximum(big, 0.01 * big)
        t = jnp.dot(h.reshape(N * N, H), w2_ref[0],
                    preferred_element_type=jnp.float32)
        t = t.reshape(N, N) + b2_ref[0]
        return jax.nn.sigmoid(t)

    c0 = expert(w10_ref, b10_ref, w20_ref, b20_ref)
    ce = expert(w1e_ref, b1e_ref, w2e_ref, b2e_ref)
    out_ref[0] = c0 * (1.0 - mask) + ce * mask


def kernel(latent, action, W_mask, b_mask, W1, b1, W2, b2):
    B, N, D = latent.shape
    A = action.shape[-1]
    H = W1.shape[-1]
    ids = jnp.argmax(action, axis=-1).astype(jnp.int32)
    pos = jnp.asarray(_pos_encoding_np(N, D))
    u = jax.random.uniform(jax.random.key(1), (B, N, 2), minval=1e-20, maxval=1.0)
    g = -jnp.log(-jnp.log(u))

    grid_spec = pltpu.PrefetchScalarGridSpec(
        num_scalar_prefetch=1,
        grid=(B,),
        in_specs=[
            pl.BlockSpec((1, N, D), lambda b, ids: (b, 0, 0)),
            pl.BlockSpec((1, 1, A), lambda b, ids: (b, 0, 0)),
            pl.BlockSpec((A + D, D), lambda b, ids: (0, 0)),
            pl.BlockSpec((1, D), lambda b, ids: (0, 0)),
            pl.BlockSpec((N, D), lambda b, ids: (0, 0)),
            pl.BlockSpec((1, N, 2), lambda b, ids: (b, 0, 0)),
            pl.BlockSpec((1, 2 * D, H), lambda b, ids: (0, 0, 0)),
            pl.BlockSpec((1, 2 * D, H), lambda b, ids: (ids[b] + 1, 0, 0)),
            pl.BlockSpec((1, 1, H), lambda b, ids: (0, 0, 0)),
            pl.BlockSpec((1, 1, H), lambda b, ids: (ids[b] + 1, 0, 0)),
            pl.BlockSpec((1, H, 1), lambda b, ids: (0, 0, 0)),
            pl.BlockSpec((1, H, 1), lambda b, ids: (ids[b] + 1, 0, 0)),
            pl.BlockSpec((1, 1, 1), lambda b, ids: (0, 0, 0)),
            pl.BlockSpec((1, 1, 1), lambda b, ids: (ids[b] + 1, 0, 0)),
        ],
        out_specs=pl.BlockSpec((1, N, N), lambda b, ids: (b, 0, 0)),
    )
    b1r = b1.reshape(A + 1, 1, H)
    b2r = b2.reshape(A + 1, 1, 1)
    return pl.pallas_call(
        _adj_body,
        grid_spec=grid_spec,
        out_shape=jax.ShapeDtypeStruct((B, N, N), jnp.float32),
        compiler_params=pltpu.CompilerParams(dimension_semantics=("arbitrary",)),
    )(ids, latent, action.reshape(B, 1, A), W_mask, b_mask.reshape(1, D), pos,
      g, W1, W1, b1r, b1r, W2, W2, b2r, b2r)


# speedup vs baseline: 5.0872x; 1.3250x over previous
import math

import jax
import jax.numpy as jnp
import numpy as np
from jax.experimental import pallas as pl
from jax.experimental.pallas import tpu as pltpu


def _pos_encoding_np(N, D):
    position = np.arange(N)[:, None].astype(np.float64)
    div_term = np.exp(np.arange(0, D, 2).astype(np.float64) * (-math.log(10000.0) / D))
    pe = np.zeros((N, D), dtype=np.float32)
    pe[:, 0::2] = np.sin(position * div_term)
    pe[:, 1::2] = np.cos(position * div_term)
    return pe


_gumbel_cache = {}


def _gumbel_const(B, N):
    if (B, N) not in _gumbel_cache:
        with jax.ensure_compile_time_eval():
            u = jax.random.uniform(jax.random.key(1), (B, N, 2),
                                   minval=1e-20, maxval=1.0)
            g = -jnp.log(-jnp.log(u))
        _gumbel_cache[(B, N)] = np.asarray(g, dtype=np.float32)
    return _gumbel_cache[(B, N)]


def _adj_body(ids_ref, latent_ref, action_ref, wmask_ref, bmask_ref, pos_ref,
              g_ref, w10_ref, w1e_ref, b1_ref, w20_ref, w2e_ref,
              b2_ref, out_ref, t0_scr, te_scr):
    N, D = latent_ref.shape[-2], latent_ref.shape[-1]
    A = action_ref.shape[-1]
    b = pl.program_id(0)
    eid = ids_ref[b] + 1
    lat = latent_ref[0]
    act = action_ref[pl.ds(b, 1), :]

    act_rep = jnp.broadcast_to(act, (N, A))
    ap = jnp.concatenate([act_rep, pos_ref[...]], axis=-1)
    inter_mask = jax.nn.sigmoid(
        jnp.dot(ap, wmask_ref[...], preferred_element_type=jnp.float32)
        + bmask_ref[...])
    s = jnp.sum(lat * inter_mask, axis=-1, keepdims=True)
    l0 = jnp.log(jnp.maximum(1.0 - s, 0.0001))
    l1 = jnp.log(jnp.maximum(s, 0.0001))
    g = g_ref[0]
    z0 = l0 + g[:, 0:1]
    z1 = l1 + g[:, 1:2]
    zm = jnp.maximum(z0, z1)
    e0 = jnp.exp(z0 - zm)
    e1 = jnp.exp(z1 - zm)
    esum = e0 + e1
    y0 = e0 / esum
    y1 = e1 / esum
    hard = (y1 > y0).astype(jnp.float32)
    mask = hard + y1 - y1

    def expert(w1_ref, b1row, w2_ref, scr):
        w1 = w1_ref[0]
        H = w1.shape[-1]
        u = jnp.dot(lat, w1[:D], preferred_element_type=jnp.float32)
        v = jnp.dot(lat, w1[D:], preferred_element_type=jnp.float32)
        big = u[:, None, :] + v[None, :, :] + b1row[None, :, :]
        h = jnp.maximum(big, 0.01 * big)
        t = jnp.dot(h.reshape(N * N, H), w2_ref[0],
                    preferred_element_type=jnp.float32)
        scr[...] = t.reshape(N, N)
        return scr[...]

    t0 = expert(w10_ref, b1_ref[0:1, :], w20_ref, t0_scr) + b2_ref[0:1, 0:1]
    te = expert(w1e_ref, b1_ref[pl.ds(eid, 1), :], w2e_ref, te_scr) \
        + b2_ref[pl.ds(eid, 1), 0:1]
    c0 = jax.nn.sigmoid(t0)
    ce = jax.nn.sigmoid(te)
    out_ref[0] = c0 * (1.0 - mask) + ce * mask


def kernel(latent, action, W_mask, b_mask, W1, b1, W2, b2):
    B, N, D = latent.shape
    A = action.shape[-1]
    H = W1.shape[-1]
    ids = jnp.argmax(action, axis=-1).astype(jnp.int32)
    pos = _pos_encoding_np(N, D)
    g = _gumbel_const(B, N)

    grid_spec = pltpu.PrefetchScalarGridSpec(
        num_scalar_prefetch=1,
        grid=(B,),
        in_specs=[
            pl.BlockSpec((1, N, D), lambda b, ids: (b, 0, 0)),
            pl.BlockSpec((B, A), lambda b, ids: (0, 0)),
            pl.BlockSpec((A + D, D), lambda b, ids: (0, 0)),
            pl.BlockSpec((1, D), lambda b, ids: (0, 0)),
            pl.BlockSpec((N, D), lambda b, ids: (0, 0)),
            pl.BlockSpec((1, N, 2), lambda b, ids: (b, 0, 0)),
            pl.BlockSpec((1, 2 * D, H), lambda b, ids: (0, 0, 0)),
            pl.BlockSpec((1, 2 * D, H), lambda b, ids: (ids[b] + 1, 0, 0)),
            pl.BlockSpec((A + 1, H), lambda b, ids: (0, 0)),
            pl.BlockSpec((1, H, 1), lambda b, ids: (0, 0, 0)),
            pl.BlockSpec((1, H, 1), lambda b, ids: (ids[b] + 1, 0, 0)),
            pl.BlockSpec((A + 1, 1), lambda b, ids: (0, 0)),
        ],
        out_specs=pl.BlockSpec((1, N, N), lambda b, ids: (b, 0, 0)),
        scratch_shapes=[pltpu.VMEM((N, N), jnp.float32),
                        pltpu.VMEM((N, N), jnp.float32)],
    )
    return pl.pallas_call(
        _adj_body,
        grid_spec=grid_spec,
        out_shape=jax.ShapeDtypeStruct((B, N, N), jnp.float32),
        compiler_params=pltpu.CompilerParams(dimension_semantics=("arbitrary",)),
    )(ids, latent, action, W_mask, b_mask.reshape(1, D), pos,
      g, W1, W1, b1, W2, W2, b2)
